# Initial kernel scaffold; baseline (speedup 1.0000x reference)
import jax
import jax.numpy as jnp
from jax.experimental import pallas as pl


def kernel(x, table):
    raise NotImplementedError("write your pallas kernel here")


# speedup vs baseline: 1.4225x; 1.4225x over previous
import functools
import math

import jax
import jax.numpy as jnp
from jax import lax
from jax.experimental import pallas as pl
from jax.experimental.pallas import tpu as pltpu
from jax.experimental.pallas import tpu_sc as plsc

_VOCAB = 1000000
_D = 32
_B = 4096
_H = 200
_N = _B * _H
_NW = 32
_PER_W = _N // _NW
_CHUNK = 1280
_NCHUNK = _PER_W // _CHUNK
_SCALE = math.sqrt(_D)

_mesh = plsc.VectorSubcoreMesh(core_axis_name="c", subcore_axis_name="s")


@functools.partial(
    pl.kernel,
    mesh=_mesh,
    out_type=jax.ShapeDtypeStruct((_N, _D), jnp.float32),
    scratch_types=[
        pltpu.VMEM((_PER_W,), jnp.int32),
        pltpu.VMEM((_CHUNK, _D), jnp.float32),
        pltpu.SemaphoreType.DMA,
    ],
    compiler_params=pltpu.CompilerParams(use_tc_tiling_on_sc=False),
)
def _emb_lookup(x_hbm, table_hbm, out_hbm, idx_v, rows_v, sem):
    wid = lax.axis_index("s") * 2 + lax.axis_index("c")
    base = wid * _PER_W
    pltpu.sync_copy(x_hbm.at[pl.ds(base, _PER_W)], idx_v)

    def chunk_body(ci, carry):
        off = ci * _CHUNK
        pltpu.async_copy(
            table_hbm.at[idx_v.at[pl.ds(off, _CHUNK)]], rows_v, sem
        ).wait()

        def scale_body(i, c2):
            r = i * 4
            for j in range(4):
                rows_v[r + j, pl.ds(0, 16)] = rows_v[r + j, pl.ds(0, 16)] * _SCALE
                rows_v[r + j, pl.ds(16, 16)] = rows_v[r + j, pl.ds(16, 16)] * _SCALE
            return c2

        lax.fori_loop(0, _CHUNK // 4, scale_body, 0)

        pltpu.sync_copy(rows_v, out_hbm.at[pl.ds(base + off, _CHUNK)])
        return carry

    lax.fori_loop(0, _NCHUNK, chunk_body, 0)


def kernel(x, table):
    xf = x.reshape(-1).astype(jnp.int32)
    out = _emb_lookup(xf, table)
    return out.reshape(_B, _H, _D)


# speedup vs baseline: 1.4791x; 1.0398x over previous
import functools
import math

import jax
import jax.numpy as jnp
from jax import lax
from jax.experimental import pallas as pl
from jax.experimental.pallas import tpu as pltpu
from jax.experimental.pallas import tpu_sc as plsc

_VOCAB = 1000000
_D = 32
_B = 4096
_H = 200
_N = _B * _H
_NW = 32
_PER_W = _N // _NW
_CHUNK = 1024
_NCHUNK = _PER_W // _CHUNK
_NBUF = 3
_SCALE = math.sqrt(_D)

_mesh = plsc.VectorSubcoreMesh(core_axis_name="c", subcore_axis_name="s")


@functools.partial(
    pl.kernel,
    mesh=_mesh,
    out_type=jax.ShapeDtypeStruct((_N, _D), jnp.float32),
    scratch_types=[
        pltpu.VMEM((_PER_W,), jnp.int32),
        [pltpu.VMEM((_CHUNK, _D), jnp.float32) for _ in range(_NBUF)],
        [pltpu.SemaphoreType.DMA for _ in range(_NBUF)],
        [pltpu.SemaphoreType.DMA for _ in range(_NBUF)],
    ],
    compiler_params=pltpu.CompilerParams(use_tc_tiling_on_sc=False),
)
def _emb_lookup(x_hbm, table_hbm, out_hbm, idx_v, rows, gsem, wsem):
    wid = lax.axis_index("s") * 2 + lax.axis_index("c")
    base = wid * _PER_W
    pltpu.sync_copy(x_hbm.at[pl.ds(base, _PER_W)], idx_v)

    def issue_gather(ci):
        return pltpu.async_copy(
            table_hbm.at[idx_v.at[pl.ds(ci * _CHUNK, _CHUNK)]],
            rows[ci % _NBUF],
            gsem[ci % _NBUF],
        )

    gathers = {}
    writes = {}
    gathers[0] = issue_gather(0)
    gathers[1] = issue_gather(1)
    for ci in range(_NCHUNK):
        b = ci % _NBUF
        gathers[ci].wait()

        buf = rows[b]

        @plsc.parallel_loop(0, _CHUNK, unroll=8)
        def _scale(i, _buf=buf):
            _buf[i, pl.ds(0, 16)] = _buf[i, pl.ds(0, 16)] * _SCALE
            _buf[i, pl.ds(16, 16)] = _buf[i, pl.ds(16, 16)] * _SCALE

        writes[ci] = pltpu.async_copy(
            buf, out_hbm.at[pl.ds(base + ci * _CHUNK, _CHUNK)], wsem[b]
        )
        nxt = ci + 2
        if nxt < _NCHUNK:
            if nxt - _NBUF >= 0:
                writes[nxt - _NBUF].wait()
            gathers[nxt] = issue_gather(nxt)
    for ci in range(max(0, _NCHUNK - _NBUF), _NCHUNK):
        writes[ci].wait()


def kernel(x, table):
    xf = x.reshape(-1).astype(jnp.int32)
    out = _emb_lookup(xf, table)
    return out.reshape(_B, _H, _D)


# speedup vs baseline: 1.5528x; 1.0498x over previous
import functools
import math

import jax
import jax.numpy as jnp
from jax import lax
from jax.experimental import pallas as pl
from jax.experimental.pallas import tpu as pltpu
from jax.experimental.pallas import tpu_sc as plsc

_VOCAB = 1000000
_D = 32
_B = 4096
_H = 200
_N = _B * _H
_NW = 32
_PER_W = _N // _NW
_CHUNK = 1024
_NCHUNK = _PER_W // _CHUNK
_NBUF = 3
_SCALE = math.sqrt(_D)

_mesh = plsc.VectorSubcoreMesh(core_axis_name="c", subcore_axis_name="s")


@functools.partial(
    pl.kernel,
    mesh=_mesh,
    out_type=jax.ShapeDtypeStruct((_N, _D), jnp.float32),
    scratch_types=[
        pltpu.VMEM((_PER_W,), jnp.int32),
        [pltpu.VMEM((_CHUNK, _D), jnp.float32) for _ in range(_NBUF)],
        [pltpu.SemaphoreType.DMA for _ in range(_NBUF)],
        [pltpu.SemaphoreType.DMA for _ in range(_NBUF)],
    ],
    compiler_params=pltpu.CompilerParams(use_tc_tiling_on_sc=False),
)
def _emb_lookup(x_hbm, table_hbm, out_hbm, idx_v, rows, gsem, wsem):
    wid = lax.axis_index("s") * 2 + lax.axis_index("c")
    base = wid * _PER_W
    pltpu.sync_copy(x_hbm.at[pl.ds(base, _PER_W)], idx_v)

    def issue_gather(ci):
        return pltpu.async_copy(
            table_hbm.at[idx_v.at[pl.ds(ci * _CHUNK, _CHUNK)]],
            rows[ci % _NBUF],
            gsem[ci % _NBUF],
        )

    gathers = {}
    writes = {}
    gathers[0] = issue_gather(0)
    gathers[1] = issue_gather(1)
    for ci in range(_NCHUNK):
        b = ci % _NBUF
        gathers[ci].wait()

        buf = rows[b]

        @plsc.parallel_loop(0, _CHUNK, unroll=8)
        def _scale(i, _buf=buf):
            _buf[i, pl.ds(0, 16)] = _buf[i, pl.ds(0, 16)] * _SCALE
            _buf[i, pl.ds(16, 16)] = _buf[i, pl.ds(16, 16)] * _SCALE

        writes[ci] = pltpu.async_copy(
            buf,
            out_hbm.at[pl.ds(base + ci * _CHUNK, _CHUNK)],
            wsem[b],
        )
        nxt = ci + 2
        if nxt < _NCHUNK:
            if nxt - _NBUF >= 0:
                writes[nxt - _NBUF].wait()
            gathers[nxt] = issue_gather(nxt)
    for ci in range(max(0, _NCHUNK - _NBUF), _NCHUNK):
        writes[ci].wait()


def kernel(x, table):
    xf = x.T.reshape(-1).astype(jnp.int32)
    out = _emb_lookup(xf, table)
    return out.reshape(_H, _B, _D).transpose(1, 0, 2)
